# baseline (speedup 1.0000x reference)
import functools

import jax
import jax.numpy as jnp
from jax import lax
from jax.experimental import pallas as pl
from jax.experimental.pallas import tpu as pltpu
from jax.experimental.pallas import tpu_sc as plsc

N = 10000
E = 160000
D = 256
H = 256
C = 40

NC = 2
NS = 16
NW = NC * NS
NPT = N // NS
HC = 128

F32 = jnp.float32


@functools.cache
def _make_sc_agg(mode, with_count, hc):
    ch = 125
    ept = (E // NS) if mode == "feat" else (E // NW)
    nch = ept // ch
    assert nch * ch == ept
    NB = 2
    NH = 40
    nhalves = nch // NH
    assert nhalves * NH == nch

    mesh = plsc.VectorSubcoreMesh(core_axis_name="c", subcore_axis_name="s",
                                  num_cores=NC, num_subcores=NS)

    if with_count:
        out_type = [jax.ShapeDtypeStruct((NC, N, hc), F32),
                    jax.ShapeDtypeStruct((N,), F32)]
    else:
        out_type = jax.ShapeDtypeStruct((NC, N, hc), F32)

    scratch = [
        pltpu.VMEM_SHARED((N, hc), F32),
        pltpu.VMEM((NH, 2, ch), jnp.int32),
        pltpu.VMEM((NB, ch, hc), F32),
        [pltpu.SemaphoreType.DMA] * NB,
        [pltpu.SemaphoreType.DMA] * NB,
    ]
    if with_count:
        scratch += [
            pltpu.VMEM_SHARED((N,), F32),
            pltpu.VMEM((ch,), F32),
            pltpu.VMEM((1000,), F32),
        ]

    def body(y_hbm, comb_hbm, zrows_hbm, zn_hbm, ones_hbm, *rest):
        if with_count:
            (out_hbm, cnt_hbm, acc, ibuf, rows, gsem, ssem, cacc,
             ones_v, cbuf) = rest
        else:
            out_hbm, acc, ibuf, rows, gsem, ssem = rest
        c = lax.axis_index("c")
        s = lax.axis_index("s")
        wid = c * NS + s

        def zero_acc():
            @pl.when(s < 10)
            def _():
                pltpu.sync_copy(zrows_hbm, acc.at[pl.ds(s * 1000, 1000)])
            if with_count:
                pltpu.sync_copy(ones_hbm, ones_v)

                @pl.when(jnp.logical_and(c == 0, s < 10))
                def _():
                    pltpu.sync_copy(zn_hbm, cbuf)
                    pltpu.sync_copy(cbuf, cacc.at[pl.ds(s * 1000, 1000)])

        tbase = s * 80 + c * 40 if mode == "edge" else wid * nch

        def start_gather(j, b):
            pltpu.async_copy(y_hbm.at[ibuf.at[j, 0]], rows.at[b], gsem[b])

        def wait_gather(j, b):
            pltpu.make_async_copy(y_hbm.at[ibuf.at[j, 0]], rows.at[b],
                                  gsem[b]).wait()

        def start_scatter(j, b):
            pltpu.async_copy(rows.at[b], acc.at[ibuf.at[j, 1]], ssem[b],
                             add=True)
            if with_count:
                @pl.when(c == 0)
                def _():
                    pltpu.sync_copy(ones_v, cacc.at[ibuf.at[j, 1]], add=True)

        def wait_scatter(j, b):
            pltpu.make_async_copy(rows.at[b], acc.at[ibuf.at[j, 1]],
                                  ssem[b]).wait()

        ngr = NH // NB
        for h in range(nhalves):
            pltpu.sync_copy(comb_hbm.at[pl.ds(tbase + h * NH, NH)], ibuf)
            for b in range(NB):
                start_gather(b, b)
            if h == 0:
                zero_acc()
                plsc.subcore_barrier()

            def group(jg, carry):
                j0 = jg * NB
                for b in range(NB):
                    @pl.when(j0 + b < NH)
                    def _(b=b):
                        wait_gather(j0 + b, b)
                        start_scatter(j0 + b, b)
                for b in range(NB):
                    @pl.when(j0 + NB + b < NH)
                    def _(b=b):
                        wait_scatter(j0 + b, b)
                        start_gather(j0 + NB + b, b)
                return carry

            lax.fori_loop(0, ngr, group, 0)

            for b in range(NB):
                wait_scatter((ngr - 1) * NB + b, b)

        plsc.subcore_barrier()

        @pl.when(s < 10)
        def _():
            pltpu.sync_copy(acc.at[pl.ds(s * 1000, 1000)],
                            out_hbm.at[c, pl.ds(s * 1000, 1000)])
        if with_count:
            @pl.when(jnp.logical_and(c == 0, s < 10))
            def _():
                pltpu.sync_copy(cacc.at[pl.ds(s * 1000, 1000)], cbuf)
                pltpu.sync_copy(cbuf, cnt_hbm.at[pl.ds(s * 1000, 1000)])

    return pl.kernel(body, out_type=out_type, mesh=mesh,
                     scratch_types=scratch)


BN = 1000
GN = N // BN


def _y1_body(h_ref, wl_ref, ys_ref):
    ys_ref[...] = jnp.dot(h_ref[...], wl_ref[...], preferred_element_type=F32)


def _z1_body(h_ref, wr_ref, b_ref, z_ref):
    z_ref[...] = (jnp.dot(h_ref[...], wr_ref[...], preferred_element_type=F32)
                  + b_ref[...])


def _relu_h(a_ref, zin_ref, cnt_ref):
    inv = 1.0 / jnp.maximum(cnt_ref[...], 1.0)
    h0 = jnp.maximum(a_ref[0] * inv + zin_ref[:, :HC], 0.0)
    h1 = jnp.maximum(a_ref[1] * inv + zin_ref[:, HC:], 0.0)
    return h0, h1


def _ymid_body(a_ref, zin_ref, cnt_ref, wl_ref, ys_ref):
    h0, h1 = _relu_h(a_ref, zin_ref, cnt_ref)
    wl = wl_ref[...]
    ys_ref[...] = (jnp.dot(h0, wl[:HC], preferred_element_type=F32)
                   + jnp.dot(h1, wl[HC:], preferred_element_type=F32))


def _zmid_body(a_ref, zin_ref, cnt_ref, wr_ref, b_ref, z_ref):
    h0, h1 = _relu_h(a_ref, zin_ref, cnt_ref)
    wr = wr_ref[...]
    z_ref[...] = (jnp.dot(h0, wr[:HC], preferred_element_type=F32)
                  + jnp.dot(h1, wr[HC:], preferred_element_type=F32)
                  + b_ref[...])


def _mid_body(a_ref, zin_ref, cnt_ref, wl_ref, wr_ref, b_ref, ys_ref, z_ref):
    h0, h1 = _relu_h(a_ref, zin_ref, cnt_ref)
    wl = wl_ref[...]
    wr = wr_ref[...]
    ys_ref[...] = (jnp.dot(h0, wl[:HC], preferred_element_type=F32)
                   + jnp.dot(h1, wl[HC:], preferred_element_type=F32))
    z_ref[...] = (jnp.dot(h0, wr[:HC], preferred_element_type=F32)
                  + jnp.dot(h1, wr[HC:], preferred_element_type=F32)
                  + b_ref[...])


def _fin_body(p_ref, zin_ref, cnt_ref, out_ref):
    inv = 1.0 / jnp.maximum(cnt_ref[...], 1.0)
    out_ref[...] = (p_ref[0][:, :64] + p_ref[1][:, :64]) * inv + zin_ref[...]


@functools.cache
def _tc_kernels(ipret=False):
    ys_spec = pl.BlockSpec((BN, HC), lambda i, c: (c * GN + i, 0))
    z_spec = pl.BlockSpec((BN, HC), lambda i, c: (i, c))
    h_spec = pl.BlockSpec((BN, D), lambda i, c: (i, 0))
    w_spec = pl.BlockSpec((D, HC), lambda i, c: (0, c))
    b_spec = pl.BlockSpec((1, HC), lambda i, c: (0, c))
    a_spec = pl.BlockSpec((2, BN, HC), lambda i, c: (0, i, 0))
    cnt_spec = pl.BlockSpec((BN, 1), lambda i, c: (i, 0))

    y1 = pl.pallas_call(
        _y1_body, grid=(GN, 2),
        in_specs=[h_spec, w_spec],
        out_specs=ys_spec,
        out_shape=jax.ShapeDtypeStruct((2 * N, HC), F32),
        interpret=ipret,
    )
    z1 = pl.pallas_call(
        _z1_body, grid=(GN, 2),
        in_specs=[h_spec, w_spec, b_spec],
        out_specs=z_spec,
        out_shape=jax.ShapeDtypeStruct((N, H), F32),
        interpret=ipret,
    )

    def make_ymid(ncb, w):
        return pl.pallas_call(
            _ymid_body, grid=(GN, ncb),
            in_specs=[a_spec, h_spec, cnt_spec,
                      pl.BlockSpec((D, w), lambda i, c: (0, c))],
            out_specs=pl.BlockSpec((BN, w), lambda i, c: (c * GN + i, 0)),
            out_shape=jax.ShapeDtypeStruct((ncb * N, w), F32),
            interpret=ipret,
        )

    def make_zmid(ncb, w):
        return pl.pallas_call(
            _zmid_body, grid=(GN, ncb),
            in_specs=[a_spec, h_spec, cnt_spec,
                      pl.BlockSpec((D, w), lambda i, c: (0, c)),
                      pl.BlockSpec((1, w), lambda i, c: (0, c))],
            out_specs=pl.BlockSpec((BN, w), lambda i, c: (i, c)),
            out_shape=jax.ShapeDtypeStruct((N, ncb * w), F32),
            interpret=ipret,
        )

    fin = pl.pallas_call(
        _fin_body, grid=(GN,),
        in_specs=[
            pl.BlockSpec((2, BN, HC), lambda i: (0, i, 0)),
            pl.BlockSpec((BN, 64), lambda i: (i, 0)),
            pl.BlockSpec((BN, 1), lambda i: (i, 0)),
        ],
        out_specs=pl.BlockSpec((BN, 64), lambda i: (i, 0)),
        out_shape=jax.ShapeDtypeStruct((N, 64), F32),
        interpret=ipret,
    )
    def make_mid(ncb, wy, wz):
        return pl.pallas_call(
            _mid_body, grid=(GN, ncb),
            in_specs=[a_spec, h_spec, cnt_spec,
                      pl.BlockSpec((D, wy), lambda i, c: (0, c)),
                      pl.BlockSpec((D, wz), lambda i, c: (0, c)),
                      pl.BlockSpec((1, wz), lambda i, c: (0, c))],
            out_specs=[pl.BlockSpec((BN, wy), lambda i, c: (c * GN + i, 0)),
                       pl.BlockSpec((BN, wz), lambda i, c: (i, c))],
            out_shape=[jax.ShapeDtypeStruct((ncb * N, wy), F32),
                       jax.ShapeDtypeStruct((N, ncb * wz), F32)],
            interpret=ipret,
        )

    return (y1, z1, make_mid(2, HC, HC), make_mid(1, HC, 64), fin)


@jax.jit
def kernel(x, edge_index, W1l, W1r, b1, W2l, W2r, b2, W3l, W3r, b3):
    src = edge_index[0]
    dst = edge_index[1]
    zrows = jnp.zeros((1000, HC), F32)
    zn = jnp.zeros((1000,), F32)
    ones125 = jnp.ones((125,), F32)

    sf = jnp.stack([src, src + N]).reshape(NC, NS, 80, 125)
    df = jnp.broadcast_to(dst.reshape(1, NS, 80, 125), (NC, NS, 80, 125))
    comb_f = jnp.stack([sf, df], axis=3).reshape(NC * NS * 80, 2, 125)

    W3lp = jnp.pad(W3l, ((0, 0), (0, HC - C)))
    W3rp = jnp.pad(W3r, ((0, 0), (0, 64 - C)))
    b3p = jnp.pad(b3, (0, 64 - C)).reshape(1, 64)

    y1, z1k, mid2, mid3, _fin = _tc_kernels()

    ys1 = y1(x, W1l)
    agg1, cnt = _make_sc_agg("feat", True, HC)(ys1, comb_f, zrows, zn, ones125)
    z1 = z1k(x, W1r, b1.reshape(1, H))
    cnt2 = cnt.reshape(N, 1)

    ys2, z2 = mid2(agg1, z1, cnt2, W2l, W2r, b2.reshape(1, H))
    agg2 = _make_sc_agg("feat", False, HC)(ys2, comb_f, zrows, zn, ones125)

    ys3, z3 = mid3(agg2, z2, cnt2, W3lp, W3rp, b3p)
    parts = _make_sc_agg("edge", False, HC)(ys3, comb_f, zrows, zn, ones125)

    outp = _fin(parts, z3, cnt2)
    return outp[:, :C]


# speedup vs baseline: 1.0458x; 1.0458x over previous
import functools

import jax
import jax.numpy as jnp
from jax import lax
from jax.experimental import pallas as pl
from jax.experimental.pallas import tpu as pltpu
from jax.experimental.pallas import tpu_sc as plsc

N = 10000
E = 160000
D = 256
H = 256
C = 40

NC = 2
NS = 16
NW = NC * NS
NPT = N // NS
HC = 128

F32 = jnp.float32


@functools.cache
def _make_sc_agg(mode, with_count, hc):
    ch = 100 if mode == "feat" else 50
    ept = (E // NS) if mode == "feat" else (E // NW)
    nch = ept // ch
    assert nch * ch == ept
    NB = 3
    NH = 25
    nstg = nch // NH
    assert nstg * NH == nch

    mesh = plsc.VectorSubcoreMesh(core_axis_name="c", subcore_axis_name="s",
                                  num_cores=NC, num_subcores=NS)

    if with_count:
        out_type = [jax.ShapeDtypeStruct((NC, N, hc), F32),
                    jax.ShapeDtypeStruct((N,), F32)]
    else:
        out_type = jax.ShapeDtypeStruct((NC, N, hc), F32)

    scratch = [
        pltpu.VMEM_SHARED((N, hc), F32),
        pltpu.VMEM((NH, ch), jnp.int32),
        pltpu.VMEM((NH, ch), jnp.int32),
        pltpu.VMEM((NB, ch, hc), F32),
        [pltpu.SemaphoreType.DMA] * NB,
        [pltpu.SemaphoreType.DMA] * NB,
    ]
    if with_count:
        scratch += [
            pltpu.VMEM_SHARED((N,), F32),
            pltpu.VMEM((ch,), F32),
            pltpu.VMEM((1000,), F32),
        ]

    def body(y_hbm, src_hbm, dst_hbm, zrows_hbm, zn_hbm, ones_hbm, *rest):
        if with_count:
            (out_hbm, cnt_hbm, acc, sbuf, dbuf, rows, gsem, ssem, cacc,
             ones_v, cbuf) = rest
        else:
            out_hbm, acc, sbuf, dbuf, rows, gsem, ssem = rest
        c = lax.axis_index("c")
        s = lax.axis_index("s")
        wid = c * NS + s

        def zero_acc():
            @pl.when(s < 10)
            def _():
                pltpu.sync_copy(zrows_hbm, acc.at[pl.ds(s * 1000, 1000)])
            if with_count:
                pltpu.sync_copy(ones_hbm, ones_v)

                @pl.when(jnp.logical_and(c == 0, s < 10))
                def _():
                    pltpu.sync_copy(zn_hbm, cbuf)
                    pltpu.sync_copy(cbuf, cacc.at[pl.ds(s * 1000, 1000)])


        def start_gather(j, b):
            pltpu.async_copy(y_hbm.at[sbuf.at[j]], rows.at[b], gsem[b])

        def wait_gather(j, b):
            pltpu.make_async_copy(y_hbm.at[sbuf.at[j]], rows.at[b],
                                  gsem[b]).wait()

        def start_scatter(j, b):
            pltpu.async_copy(rows.at[b], acc.at[dbuf.at[j]], ssem[b],
                             add=True)
            if with_count:
                @pl.when(c == 0)
                def _():
                    pltpu.sync_copy(ones_v, cacc.at[dbuf.at[j]], add=True)

        def wait_scatter(j, b):
            pltpu.make_async_copy(rows.at[b], acc.at[dbuf.at[j]],
                                  ssem[b]).wait()

        ngr = (NH + NB - 1) // NB
        for h in range(nstg):
            pltpu.sync_copy(src_hbm.at[wid, h], sbuf)
            if mode == "feat":
                pltpu.sync_copy(dst_hbm.at[s, h], dbuf)
            else:
                pltpu.sync_copy(dst_hbm.at[wid, h], dbuf)
            for b in range(NB):
                start_gather(b, b)
            if h == 0:
                zero_acc()
                plsc.subcore_barrier()

            def group(jg, carry):
                j0 = jg * NB
                for b in range(NB):
                    @pl.when(j0 + b < NH)
                    def _(b=b):
                        wait_gather(j0 + b, b)
                        start_scatter(j0 + b, b)
                for b in range(NB):
                    @pl.when(j0 + NB + b < NH)
                    def _(b=b):
                        wait_scatter(j0 + b, b)
                        start_gather(j0 + NB + b, b)
                return carry

            lax.fori_loop(0, ngr, group, 0)

            for b in range(NB):
                jb = ((NH - 1 - b) // NB) * NB + b
                wait_scatter(jb, b)

        plsc.subcore_barrier()

        @pl.when(s < 10)
        def _():
            pltpu.sync_copy(acc.at[pl.ds(s * 1000, 1000)],
                            out_hbm.at[c, pl.ds(s * 1000, 1000)])
        if with_count:
            @pl.when(jnp.logical_and(c == 0, s < 10))
            def _():
                pltpu.sync_copy(cacc.at[pl.ds(s * 1000, 1000)], cbuf)
                pltpu.sync_copy(cbuf, cnt_hbm.at[pl.ds(s * 1000, 1000)])

    return pl.kernel(body, out_type=out_type, mesh=mesh,
                     scratch_types=scratch)


BN = 1000
GN = N // BN


def _y1_body(h_ref, wl_ref, ys_ref):
    ys_ref[...] = jnp.dot(h_ref[...], wl_ref[...], preferred_element_type=F32)


def _z1_body(h_ref, wr_ref, b_ref, z_ref):
    z_ref[...] = (jnp.dot(h_ref[...], wr_ref[...], preferred_element_type=F32)
                  + b_ref[...])


def _relu_h(a_ref, zin_ref, cnt_ref):
    inv = 1.0 / jnp.maximum(cnt_ref[...], 1.0)
    h0 = jnp.maximum(a_ref[0] * inv + zin_ref[:, :HC], 0.0)
    h1 = jnp.maximum(a_ref[1] * inv + zin_ref[:, HC:], 0.0)
    return h0, h1


def _ymid_body(a_ref, zin_ref, cnt_ref, wl_ref, ys_ref):
    h0, h1 = _relu_h(a_ref, zin_ref, cnt_ref)
    wl = wl_ref[...]
    ys_ref[...] = (jnp.dot(h0, wl[:HC], preferred_element_type=F32)
                   + jnp.dot(h1, wl[HC:], preferred_element_type=F32))


def _zmid_body(a_ref, zin_ref, cnt_ref, wr_ref, b_ref, z_ref):
    h0, h1 = _relu_h(a_ref, zin_ref, cnt_ref)
    wr = wr_ref[...]
    z_ref[...] = (jnp.dot(h0, wr[:HC], preferred_element_type=F32)
                  + jnp.dot(h1, wr[HC:], preferred_element_type=F32)
                  + b_ref[...])


def _mid_body(a_ref, zin_ref, cnt_ref, wl_ref, wr_ref, b_ref, ys_ref, z_ref):
    h0, h1 = _relu_h(a_ref, zin_ref, cnt_ref)
    wl = wl_ref[...]
    wr = wr_ref[...]
    ys_ref[...] = (jnp.dot(h0, wl[:HC], preferred_element_type=F32)
                   + jnp.dot(h1, wl[HC:], preferred_element_type=F32))
    z_ref[...] = (jnp.dot(h0, wr[:HC], preferred_element_type=F32)
                  + jnp.dot(h1, wr[HC:], preferred_element_type=F32)
                  + b_ref[...])


def _fin_body(p_ref, zin_ref, cnt_ref, out_ref):
    inv = 1.0 / jnp.maximum(cnt_ref[...], 1.0)
    out_ref[...] = (p_ref[0][:, :64] + p_ref[1][:, :64]) * inv + zin_ref[...]


@functools.cache
def _tc_kernels(ipret=False):
    ys_spec = pl.BlockSpec((BN, HC), lambda i, c: (c * GN + i, 0))
    z_spec = pl.BlockSpec((BN, HC), lambda i, c: (i, c))
    h_spec = pl.BlockSpec((BN, D), lambda i, c: (i, 0))
    w_spec = pl.BlockSpec((D, HC), lambda i, c: (0, c))
    b_spec = pl.BlockSpec((1, HC), lambda i, c: (0, c))
    a_spec = pl.BlockSpec((2, BN, HC), lambda i, c: (0, i, 0))
    cnt_spec = pl.BlockSpec((BN, 1), lambda i, c: (i, 0))

    y1 = pl.pallas_call(
        _y1_body, grid=(GN, 2),
        in_specs=[h_spec, w_spec],
        out_specs=ys_spec,
        out_shape=jax.ShapeDtypeStruct((2 * N, HC), F32),
        interpret=ipret,
    )
    z1 = pl.pallas_call(
        _z1_body, grid=(GN, 2),
        in_specs=[h_spec, w_spec, b_spec],
        out_specs=z_spec,
        out_shape=jax.ShapeDtypeStruct((N, H), F32),
        interpret=ipret,
    )

    def make_ymid(ncb, w):
        return pl.pallas_call(
            _ymid_body, grid=(GN, ncb),
            in_specs=[a_spec, h_spec, cnt_spec,
                      pl.BlockSpec((D, w), lambda i, c: (0, c))],
            out_specs=pl.BlockSpec((BN, w), lambda i, c: (c * GN + i, 0)),
            out_shape=jax.ShapeDtypeStruct((ncb * N, w), F32),
            interpret=ipret,
        )

    def make_zmid(ncb, w):
        return pl.pallas_call(
            _zmid_body, grid=(GN, ncb),
            in_specs=[a_spec, h_spec, cnt_spec,
                      pl.BlockSpec((D, w), lambda i, c: (0, c)),
                      pl.BlockSpec((1, w), lambda i, c: (0, c))],
            out_specs=pl.BlockSpec((BN, w), lambda i, c: (i, c)),
            out_shape=jax.ShapeDtypeStruct((N, ncb * w), F32),
            interpret=ipret,
        )

    fin = pl.pallas_call(
        _fin_body, grid=(GN,),
        in_specs=[
            pl.BlockSpec((2, BN, HC), lambda i: (0, i, 0)),
            pl.BlockSpec((BN, 64), lambda i: (i, 0)),
            pl.BlockSpec((BN, 1), lambda i: (i, 0)),
        ],
        out_specs=pl.BlockSpec((BN, 64), lambda i: (i, 0)),
        out_shape=jax.ShapeDtypeStruct((N, 64), F32),
        interpret=ipret,
    )
    def make_mid(ncb, wy, wz):
        return pl.pallas_call(
            _mid_body, grid=(GN, ncb),
            in_specs=[a_spec, h_spec, cnt_spec,
                      pl.BlockSpec((D, wy), lambda i, c: (0, c)),
                      pl.BlockSpec((D, wz), lambda i, c: (0, c)),
                      pl.BlockSpec((1, wz), lambda i, c: (0, c))],
            out_specs=[pl.BlockSpec((BN, wy), lambda i, c: (c * GN + i, 0)),
                       pl.BlockSpec((BN, wz), lambda i, c: (i, c))],
            out_shape=[jax.ShapeDtypeStruct((ncb * N, wy), F32),
                       jax.ShapeDtypeStruct((N, ncb * wz), F32)],
            interpret=ipret,
        )

    return (y1, z1, make_ymid(2, HC), make_zmid(2, HC),
            make_ymid(1, HC), make_zmid(1, 64), fin)


@jax.jit
def kernel(x, edge_index, W1l, W1r, b1, W2l, W2r, b2, W3l, W3r, b3):
    src = edge_index[0]
    dst = edge_index[1]
    zrows = jnp.zeros((1000, HC), F32)
    zn = jnp.zeros((1000,), F32)
    ones = jnp.ones((100,), F32)

    srcf = jnp.concatenate([src, src + N]).reshape(NW, 4, 25, 100)
    dstf = dst.reshape(NS, 4, 25, 100)
    srce = src.reshape(NW, 4, 25, 50)
    dste_e = dst.reshape(NW, 4, 25, 50)

    W3lp = jnp.pad(W3l, ((0, 0), (0, HC - C)))
    W3rp = jnp.pad(W3r, ((0, 0), (0, 64 - C)))
    b3p = jnp.pad(b3, (0, 64 - C)).reshape(1, 64)

    y1, z1k, y2k, z2k, y3k, z3k, _fin = _tc_kernels()

    ys1 = y1(x, W1l)
    agg1, cnt = _make_sc_agg("feat", True, HC)(ys1, srcf, dstf, zrows, zn, ones)
    z1 = z1k(x, W1r, b1.reshape(1, H))
    cnt2 = cnt.reshape(N, 1)

    ys2 = y2k(agg1, z1, cnt2, W2l)
    agg2 = _make_sc_agg("feat", False, HC)(ys2, srcf, dstf, zrows, zn, ones)
    z2 = z2k(agg1, z1, cnt2, W2r, b2.reshape(1, H))

    ys3 = y3k(agg2, z2, cnt2, W3lp)
    parts = _make_sc_agg("edge", False, HC)(ys3, srce, dste_e, zrows, zn, ones)
    z3 = z3k(agg2, z2, cnt2, W3rp, b3p)

    outp = _fin(parts, z3, cnt2)
    return outp[:, :C]


# speedup vs baseline: 1.0676x; 1.0208x over previous
import functools

import jax
import jax.numpy as jnp
from jax import lax
from jax.experimental import pallas as pl
from jax.experimental.pallas import tpu as pltpu
from jax.experimental.pallas import tpu_sc as plsc

N = 10000
E = 160000
D = 256
H = 256
C = 40

NC = 2
NS = 16
NW = NC * NS
NPT = N // NS
HC = 128

F32 = jnp.float32


@functools.cache
def _make_sc_agg(mode, with_count, hc):
    ch = 100
    ept = (E // NS) if mode == "feat" else (E // NW)
    nch = ept // ch
    assert nch * ch == ept
    NB = 3
    NH = 25
    nstg = nch // NH
    assert nstg * NH == nch

    mesh = plsc.VectorSubcoreMesh(core_axis_name="c", subcore_axis_name="s",
                                  num_cores=NC, num_subcores=NS)

    if with_count:
        out_type = [jax.ShapeDtypeStruct((NC, N, hc), F32),
                    jax.ShapeDtypeStruct((N,), F32)]
    else:
        out_type = jax.ShapeDtypeStruct((NC, N, hc), F32)

    scratch = [
        pltpu.VMEM_SHARED((N, hc), F32),
        pltpu.VMEM((NH, ch), jnp.int32),
        pltpu.VMEM((NH, ch), jnp.int32),
        pltpu.VMEM((NB, ch, hc), F32),
        [pltpu.SemaphoreType.DMA] * NB,
        [pltpu.SemaphoreType.DMA] * NB,
    ]
    if with_count:
        scratch += [
            pltpu.VMEM_SHARED((N,), F32),
            pltpu.VMEM((ch,), F32),
            pltpu.VMEM((1000,), F32),
        ]

    def body(y_hbm, src_hbm, dst_hbm, zrows_hbm, zn_hbm, ones_hbm, *rest):
        if with_count:
            (out_hbm, cnt_hbm, acc, sbuf, dbuf, rows, gsem, ssem, cacc,
             ones_v, cbuf) = rest
        else:
            out_hbm, acc, sbuf, dbuf, rows, gsem, ssem = rest
        c = lax.axis_index("c")
        s = lax.axis_index("s")
        wid = c * NS + s

        def zero_acc():
            @pl.when(s < 10)
            def _():
                pltpu.sync_copy(zrows_hbm, acc.at[pl.ds(s * 1000, 1000)])
            if with_count:
                pltpu.sync_copy(ones_hbm, ones_v)

                @pl.when(jnp.logical_and(c == 0, s < 10))
                def _():
                    pltpu.sync_copy(zn_hbm, cbuf)
                    pltpu.sync_copy(cbuf, cacc.at[pl.ds(s * 1000, 1000)])


        def start_gather(j, b):
            pltpu.async_copy(y_hbm.at[sbuf.at[j]], rows.at[b], gsem[b])

        def wait_gather(j, b):
            pltpu.make_async_copy(y_hbm.at[sbuf.at[j]], rows.at[b],
                                  gsem[b]).wait()

        def start_scatter(j, b):
            pltpu.async_copy(rows.at[b], acc.at[dbuf.at[j]], ssem[b],
                             add=True)
            if with_count:
                @pl.when(c == 0)
                def _():
                    pltpu.sync_copy(ones_v, cacc.at[dbuf.at[j]], add=True)

        def wait_scatter(j, b):
            pltpu.make_async_copy(rows.at[b], acc.at[dbuf.at[j]],
                                  ssem[b]).wait()

        ngr = (NH + NB - 1) // NB
        for h in range(nstg):
            pltpu.sync_copy(src_hbm.at[wid, h], sbuf)
            if mode == "feat":
                pltpu.sync_copy(dst_hbm.at[s, h], dbuf)
            else:
                pltpu.sync_copy(dst_hbm.at[wid, h], dbuf)
            for b in range(NB):
                start_gather(b, b)
            if h == 0:
                zero_acc()
                plsc.subcore_barrier()

            def group(jg, carry):
                j0 = jg * NB
                for b in range(NB):
                    @pl.when(j0 + b < NH)
                    def _(b=b):
                        wait_gather(j0 + b, b)
                        start_scatter(j0 + b, b)
                for b in range(NB):
                    @pl.when(j0 + NB + b < NH)
                    def _(b=b):
                        wait_scatter(j0 + b, b)
                        start_gather(j0 + NB + b, b)
                return carry

            lax.fori_loop(0, ngr, group, 0)

            for b in range(NB):
                jb = ((NH - 1 - b) // NB) * NB + b
                wait_scatter(jb, b)

        plsc.subcore_barrier()

        @pl.when(s < 10)
        def _():
            pltpu.sync_copy(acc.at[pl.ds(s * 1000, 1000)],
                            out_hbm.at[c, pl.ds(s * 1000, 1000)])
        if with_count:
            @pl.when(jnp.logical_and(c == 0, s < 10))
            def _():
                pltpu.sync_copy(cacc.at[pl.ds(s * 1000, 1000)], cbuf)
                pltpu.sync_copy(cbuf, cnt_hbm.at[pl.ds(s * 1000, 1000)])

    return pl.kernel(body, out_type=out_type, mesh=mesh,
                     scratch_types=scratch)


BN = 1000
GN = N // BN


def _y1_body(h_ref, wl_ref, ys_ref):
    ys_ref[...] = jnp.dot(h_ref[...], wl_ref[...], preferred_element_type=F32)


def _z1_body(h_ref, wr_ref, b_ref, z_ref):
    z_ref[...] = (jnp.dot(h_ref[...], wr_ref[...], preferred_element_type=F32)
                  + b_ref[...])


def _relu_h(a_ref, zin_ref, cnt_ref):
    inv = 1.0 / jnp.maximum(cnt_ref[...], 1.0)
    h0 = jnp.maximum(a_ref[0] * inv + zin_ref[:, :HC], 0.0)
    h1 = jnp.maximum(a_ref[1] * inv + zin_ref[:, HC:], 0.0)
    return h0, h1


def _ymid_body(a_ref, zin_ref, cnt_ref, wl_ref, ys_ref):
    h0, h1 = _relu_h(a_ref, zin_ref, cnt_ref)
    wl = wl_ref[...]
    ys_ref[...] = (jnp.dot(h0, wl[:HC], preferred_element_type=F32)
                   + jnp.dot(h1, wl[HC:], preferred_element_type=F32))


def _zmid_body(a_ref, zin_ref, cnt_ref, wr_ref, b_ref, z_ref):
    h0, h1 = _relu_h(a_ref, zin_ref, cnt_ref)
    wr = wr_ref[...]
    z_ref[...] = (jnp.dot(h0, wr[:HC], preferred_element_type=F32)
                  + jnp.dot(h1, wr[HC:], preferred_element_type=F32)
                  + b_ref[...])


def _mid_body(a_ref, zin_ref, cnt_ref, wl_ref, wr_ref, b_ref, ys_ref, z_ref):
    h0, h1 = _relu_h(a_ref, zin_ref, cnt_ref)
    wl = wl_ref[...]
    wr = wr_ref[...]
    ys_ref[...] = (jnp.dot(h0, wl[:HC], preferred_element_type=F32)
                   + jnp.dot(h1, wl[HC:], preferred_element_type=F32))
    z_ref[...] = (jnp.dot(h0, wr[:HC], preferred_element_type=F32)
                  + jnp.dot(h1, wr[HC:], preferred_element_type=F32)
                  + b_ref[...])


def _fin_body(p_ref, zin_ref, cnt_ref, out_ref):
    inv = 1.0 / jnp.maximum(cnt_ref[...], 1.0)
    out_ref[...] = (p_ref[0][:, :64] + p_ref[1][:, :64]) * inv + zin_ref[...]


@functools.cache
def _tc_kernels(ipret=False):
    ys_spec = pl.BlockSpec((BN, HC), lambda i, c: (c * GN + i, 0))
    z_spec = pl.BlockSpec((BN, HC), lambda i, c: (i, c))
    h_spec = pl.BlockSpec((BN, D), lambda i, c: (i, 0))
    w_spec = pl.BlockSpec((D, HC), lambda i, c: (0, c))
    b_spec = pl.BlockSpec((1, HC), lambda i, c: (0, c))
    a_spec = pl.BlockSpec((2, BN, HC), lambda i, c: (0, i, 0))
    cnt_spec = pl.BlockSpec((BN, 1), lambda i, c: (i, 0))

    y1 = pl.pallas_call(
        _y1_body, grid=(GN, 2),
        in_specs=[h_spec, w_spec],
        out_specs=ys_spec,
        out_shape=jax.ShapeDtypeStruct((2 * N, HC), F32),
        interpret=ipret,
    )
    z1 = pl.pallas_call(
        _z1_body, grid=(GN, 2),
        in_specs=[h_spec, w_spec, b_spec],
        out_specs=z_spec,
        out_shape=jax.ShapeDtypeStruct((N, H), F32),
        interpret=ipret,
    )

    def make_ymid(ncb, w):
        return pl.pallas_call(
            _ymid_body, grid=(GN, ncb),
            in_specs=[a_spec, h_spec, cnt_spec,
                      pl.BlockSpec((D, w), lambda i, c: (0, c))],
            out_specs=pl.BlockSpec((BN, w), lambda i, c: (c * GN + i, 0)),
            out_shape=jax.ShapeDtypeStruct((ncb * N, w), F32),
            interpret=ipret,
        )

    def make_zmid(ncb, w):
        return pl.pallas_call(
            _zmid_body, grid=(GN, ncb),
            in_specs=[a_spec, h_spec, cnt_spec,
                      pl.BlockSpec((D, w), lambda i, c: (0, c)),
                      pl.BlockSpec((1, w), lambda i, c: (0, c))],
            out_specs=pl.BlockSpec((BN, w), lambda i, c: (i, c)),
            out_shape=jax.ShapeDtypeStruct((N, ncb * w), F32),
            interpret=ipret,
        )

    fin = pl.pallas_call(
        _fin_body, grid=(GN,),
        in_specs=[
            pl.BlockSpec((2, BN, HC), lambda i: (0, i, 0)),
            pl.BlockSpec((BN, 64), lambda i: (i, 0)),
            pl.BlockSpec((BN, 1), lambda i: (i, 0)),
        ],
        out_specs=pl.BlockSpec((BN, 64), lambda i: (i, 0)),
        out_shape=jax.ShapeDtypeStruct((N, 64), F32),
        interpret=ipret,
    )
    def make_mid(ncb, wy, wz):
        return pl.pallas_call(
            _mid_body, grid=(GN, ncb),
            in_specs=[a_spec, h_spec, cnt_spec,
                      pl.BlockSpec((D, wy), lambda i, c: (0, c)),
                      pl.BlockSpec((D, wz), lambda i, c: (0, c)),
                      pl.BlockSpec((1, wz), lambda i, c: (0, c))],
            out_specs=[pl.BlockSpec((BN, wy), lambda i, c: (c * GN + i, 0)),
                       pl.BlockSpec((BN, wz), lambda i, c: (i, c))],
            out_shape=[jax.ShapeDtypeStruct((ncb * N, wy), F32),
                       jax.ShapeDtypeStruct((N, ncb * wz), F32)],
            interpret=ipret,
        )

    return (y1, z1, make_ymid(2, HC), make_zmid(2, HC),
            make_ymid(1, HC), make_zmid(1, 64), fin)


@jax.jit
def kernel(x, edge_index, W1l, W1r, b1, W2l, W2r, b2, W3l, W3r, b3):
    src = edge_index[0]
    dst = edge_index[1]
    zrows = jnp.zeros((1000, HC), F32)
    zn = jnp.zeros((1000,), F32)
    ones = jnp.ones((100,), F32)

    srcf = jnp.concatenate([src, src + N]).reshape(NW, 4, 25, 100)
    dstf = dst.reshape(NS, 4, 25, 100)
    srce = src.reshape(NW, 2, 25, 100)
    dste_e = dst.reshape(NW, 2, 25, 100)

    W3lp = jnp.pad(W3l, ((0, 0), (0, HC - C)))
    W3rp = jnp.pad(W3r, ((0, 0), (0, 64 - C)))
    b3p = jnp.pad(b3, (0, 64 - C)).reshape(1, 64)

    y1, z1k, y2k, z2k, y3k, z3k, _fin = _tc_kernels()

    ys1 = y1(x, W1l)
    agg1, cnt = _make_sc_agg("feat", True, HC)(ys1, srcf, dstf, zrows, zn, ones)
    z1 = z1k(x, W1r, b1.reshape(1, H))
    cnt2 = cnt.reshape(N, 1)

    ys2 = y2k(agg1, z1, cnt2, W2l)
    agg2 = _make_sc_agg("feat", False, HC)(ys2, srcf, dstf, zrows, zn, ones)
    z2 = z2k(agg1, z1, cnt2, W2r, b2.reshape(1, H))

    ys3 = y3k(agg2, z2, cnt2, W3lp)
    parts = _make_sc_agg("edge", False, HC)(ys3, srce, dste_e, zrows, zn, ones)
    z3 = z3k(agg2, z2, cnt2, W3rp, b3p)

    outp = _fin(parts, z3, cnt2)
    return outp[:, :C]
